# Initial kernel scaffold; baseline (speedup 1.0000x reference)
import jax
import jax.numpy as jnp
from jax.experimental import pallas as pl


def kernel(input, slice_idx_mat):
    raise NotImplementedError("write your pallas kernel here")


# speedup vs baseline: 1992.9008x; 1992.9008x over previous
import functools

import jax
import jax.numpy as jnp
from jax import lax
from jax.experimental import pallas as pl
from jax.experimental.pallas import tpu as pltpu
from jax.experimental.pallas import tpu_sc as plsc


def _make_sc_gather(B, C, S, P):
    info = plsc.get_sparse_core_info()
    NC, NS, L = info.num_cores, info.num_subcores, info.num_lanes
    NW = NC * NS
    CB = 16
    PW = 1024
    NCB = C // CB
    NT = B * NCB
    TPW = NT // NW
    NWIN = P // PW

    mesh = plsc.VectorSubcoreMesh(core_axis_name="c", subcore_axis_name="s")

    @functools.partial(
        pl.kernel,
        out_type=jax.ShapeDtypeStruct((B, C, P), jnp.float32),
        mesh=mesh,
        compiler_params=pltpu.CompilerParams(
            use_tc_tiling_on_sc=False, needs_layout_passes=False),
        scratch_types=[
            pltpu.VMEM((CB, S), jnp.float32),
            pltpu.VMEM((2, PW), jnp.int32),
            pltpu.VMEM((2, CB, PW), jnp.float32),
            pltpu.SemaphoreType.DMA,
            pltpu.SemaphoreType.DMA,
            pltpu.SemaphoreType.DMA,
            pltpu.SemaphoreType.DMA,
        ],
    )
    def sc_gather(x_hbm, idx_hbm, out_hbm, xv, idxv, outv,
                  sem_i0, sem_i1, sem_o0, sem_o1):
        wid = lax.axis_index("s") * NC + lax.axis_index("c")
        sem_i = [sem_i0, sem_i1]
        sem_o = [sem_o0, sem_o1]

        def compute_window(s):
            def body(pc, carry):
                iv = idxv[s, pl.ds(pc * L, L)]
                for c in range(CB):
                    vals = plsc.load_gather(
                        xv, [jnp.full((L,), c, jnp.int32), iv])
                    outv[s, c, pl.ds(pc * L, L)] = vals
                return carry
            lax.fori_loop(0, PW // L, body, 0, unroll=2)

        for t in range(TPW):
            task = wid + NW * t
            b = task // NCB
            cb = task % NCB

            def issue_idx(w, s):
                return pltpu.async_copy(
                    idx_hbm.at[b, pl.ds(w * PW, PW)], idxv.at[s], sem_i[s])

            def wait_idx(s):
                pltpu.make_async_copy(
                    idx_hbm.at[b, pl.ds(0, PW)], idxv.at[s], sem_i[s]).wait()

            def issue_out(w, s):
                return pltpu.async_copy(
                    outv.at[s],
                    out_hbm.at[b, pl.ds(cb * CB, CB), pl.ds(w * PW, PW)],
                    sem_o[s])

            def wait_out(s):
                pltpu.make_async_copy(
                    outv.at[s],
                    out_hbm.at[b, pl.ds(cb * CB, CB), pl.ds(0, PW)],
                    sem_o[s]).wait()

            pltpu.sync_copy(x_hbm.at[b, pl.ds(cb * CB, CB), :], xv)
            issue_idx(0, 0)
            issue_idx(1, 1)
            for w in range(2):
                wait_idx(w)
                compute_window(w)
                issue_idx(w + 2, w)
                issue_out(w, w)

            def pair_body(i, carry):
                w2 = 2 * i
                for s in range(2):
                    w = w2 + s
                    wait_idx(s)
                    wait_out(s)
                    compute_window(s)
                    issue_idx(jnp.minimum(w + 2, NWIN - 1), s)
                    issue_out(w, s)
                return carry
            lax.fori_loop(1, NWIN // 2, pair_body, 0)

            for s in range(2):
                wait_idx(s)
                wait_out(s)

    return sc_gather


def kernel(input, slice_idx_mat):
    B, C, S, _ = input.shape
    P = slice_idx_mat.shape[1]
    x = input.reshape(B, C, S)
    idx = slice_idx_mat.astype(jnp.int32)
    out = _make_sc_gather(B, C, S, P)(x, idx)
    return out[..., None]


# speedup vs baseline: 6179.7721x; 3.1009x over previous
import functools

import jax
import jax.numpy as jnp
from jax import lax
from jax.experimental import pallas as pl
from jax.experimental.pallas import tpu as pltpu
from jax.experimental.pallas import tpu_sc as plsc


def _make_sc_gather(B, C, S, P):
    info = plsc.get_sparse_core_info()
    NC, NS, L = info.num_cores, info.num_subcores, info.num_lanes
    NW = NC * NS
    CB = 16
    PW = 1024
    NCB = C // CB
    NT = B * NCB
    TPW = NT // NW
    NWIN = P // PW

    mesh = plsc.VectorSubcoreMesh(core_axis_name="c", subcore_axis_name="s")

    @functools.partial(
        pl.kernel,
        out_type=jax.ShapeDtypeStruct((B, C, P), jnp.float32),
        mesh=mesh,
        compiler_params=pltpu.CompilerParams(
            use_tc_tiling_on_sc=False, needs_layout_passes=False),
        scratch_types=[
            pltpu.VMEM((CB, S), jnp.float32),
            pltpu.VMEM((2, PW), jnp.int32),
            pltpu.VMEM((2, CB, PW), jnp.float32),
            pltpu.SemaphoreType.DMA,
            pltpu.SemaphoreType.DMA,
            pltpu.SemaphoreType.DMA,
            pltpu.SemaphoreType.DMA,
        ],
    )
    def sc_gather(x_hbm, idx_hbm, out_hbm, xv, idxv, outv,
                  sem_i0, sem_i1, sem_o0, sem_o1):
        wid = lax.axis_index("s") * NC + lax.axis_index("c")
        sem_i = [sem_i0, sem_i1]
        sem_o = [sem_o0, sem_o1]

        def compute_window(s):
            @plsc.parallel_loop(0, PW // L, unroll=2)
            def body(pc):
                iv = idxv[s, pl.ds(pc * L, L)]
                for c in range(CB):
                    vals = plsc.load_gather(
                        xv, [jnp.full((L,), c, jnp.int32), iv])
                    outv[s, c, pl.ds(pc * L, L)] = vals

        for t in range(TPW):
            task = wid + NW * t
            b = task // NCB
            cb = task % NCB

            def issue_idx(w, s):
                return pltpu.async_copy(
                    idx_hbm.at[b, pl.ds(w * PW, PW)], idxv.at[s], sem_i[s])

            def wait_idx(s):
                pltpu.make_async_copy(
                    idx_hbm.at[b, pl.ds(0, PW)], idxv.at[s], sem_i[s]).wait()

            def issue_out(w, s):
                return pltpu.async_copy(
                    outv.at[s],
                    out_hbm.at[b, pl.ds(cb * CB, CB), pl.ds(w * PW, PW)],
                    sem_o[s])

            def wait_out(s):
                pltpu.make_async_copy(
                    outv.at[s],
                    out_hbm.at[b, pl.ds(cb * CB, CB), pl.ds(0, PW)],
                    sem_o[s]).wait()

            pltpu.sync_copy(x_hbm.at[b, pl.ds(cb * CB, CB), :], xv)
            issue_idx(0, 0)
            issue_idx(1, 1)
            for w in range(2):
                wait_idx(w)
                compute_window(w)
                issue_idx(w + 2, w)
                issue_out(w, w)

            def pair_body(i, carry):
                w2 = 2 * i
                for s in range(2):
                    w = w2 + s
                    wait_idx(s)
                    wait_out(s)
                    compute_window(s)
                    issue_idx(jnp.minimum(w + 2, NWIN - 1), s)
                    issue_out(w, s)
                return carry
            lax.fori_loop(1, NWIN // 2, pair_body, 0)

            for s in range(2):
                wait_idx(s)
                wait_out(s)

    return sc_gather


def kernel(input, slice_idx_mat):
    B, C, S, _ = input.shape
    P = slice_idx_mat.shape[1]
    x = input.reshape(B, C, S)
    idx = slice_idx_mat.astype(jnp.int32)
    out = _make_sc_gather(B, C, S, P)(x, idx)
    return out[..., None]
